# baseline (speedup 1.0000x reference)
import functools

import jax
import jax.numpy as jnp
from jax import lax
from jax.experimental import pallas as pl
from jax.experimental.pallas import tpu as pltpu
from jax.experimental.pallas import tpu_sc as plsc

NC = 2
NS = 16
LANES = 16

ROW_BLK = 512


def _mesh():
    return plsc.VectorSubcoreMesh(core_axis_name="c", subcore_axis_name="s")


def _make_deg_kernel(E, NDEG, K):
    e_per_tile = E // (NC * NS)
    n_chunks = e_per_tile // K
    per_tile_n = NDEG // NS
    zpad = ((per_tile_n + LANES - 1) // LANES) * LANES

    def body(dst_hbm, out_hbm, dst_buf, ones_v, zvec, acc, sem):
        c = lax.axis_index("c")
        s = lax.axis_index("s")
        wid = c * NS + s

        pltpu.async_copy(dst_hbm.at[wid], dst_buf, sem)

        def zb(i, _):
            zvec[pl.ds(i * LANES, LANES)] = jnp.zeros((LANES,), jnp.float32)
            return 0
        lax.fori_loop(0, zpad // LANES, zb, 0)
        pltpu.sync_copy(zvec.at[pl.ds(0, per_tile_n)],
                        acc.at[pl.ds(s * per_tile_n, per_tile_n)])

        def ob(i, _):
            ones_v[pl.ds(i * LANES, LANES)] = jnp.ones((LANES,), jnp.float32)
            return 0
        lax.fori_loop(0, K // LANES, ob, 0)

        pltpu.make_async_copy(dst_hbm.at[wid], dst_buf, sem).wait()
        plsc.subcore_barrier()

        def chunk(i, _):
            pltpu.sync_copy(ones_v, acc.at[dst_buf.at[i]], add=True)
            return 0
        lax.fori_loop(0, n_chunks, chunk, 0)

        plsc.subcore_barrier()
        pltpu.sync_copy(acc.at[pl.ds(s * per_tile_n, per_tile_n)],
                        out_hbm.at[c, pl.ds(s * per_tile_n, per_tile_n)])

    return pl.kernel(
        body,
        out_type=jax.ShapeDtypeStruct((NC, NDEG), jnp.float32),
        mesh=_mesh(),
        scratch_types=[
            pltpu.VMEM((n_chunks, K), jnp.int32),
            pltpu.VMEM((K,), jnp.float32),
            pltpu.VMEM((zpad,), jnp.float32),
            pltpu.VMEM_SHARED((NDEG,), jnp.float32),
            pltpu.SemaphoreType.DMA,
        ],
    )


def _make_agg_kernel(E, D, K, N_ACC):
    e_per_tile = E // (NC * NS)
    n_chunks = e_per_tile // K
    per_tile_n = N_ACC // NS
    ZR = 8
    NB = 4

    def body(y_hbm, src_hbm, dst_hbm, out_hbm,
             src_is, dst_is, rows, zbuf, acc, isems, gsems):
        c = lax.axis_index("c")
        s = lax.axis_index("s")
        wid = c * NS + s
        ebase = wid * (n_chunks * K)

        def idx_issue(i, p):
            pltpu.async_copy(src_hbm.at[pl.ds(ebase + i * K, K)],
                             src_is.at[pl.ds(p * K, K)], isems[p])
            pltpu.async_copy(dst_hbm.at[pl.ds(ebase + i * K, K)],
                             dst_is.at[p], isems[p])

        def idx_wait(i, p):
            pltpu.make_async_copy(src_hbm.at[pl.ds(ebase + i * K, K)],
                                  src_is.at[pl.ds(p * K, K)], isems[p]).wait()
            pltpu.make_async_copy(dst_hbm.at[pl.ds(ebase + i * K, K)],
                                  dst_is.at[p], isems[p]).wait()

        def gissue(i, p):
            pltpu.async_copy(y_hbm.at[src_is.at[pl.ds(p * K, K)]],
                             rows.at[p], gsems[p])

        def gwait(p):
            pltpu.make_async_copy(y_hbm.at[src_is.at[pl.ds(p * K, K)]],
                                  rows.at[p], gsems[p]).wait()

        def scat(p):
            pltpu.sync_copy(rows.at[p], acc.at[dst_is.at[p]], add=True)

        for i in range(3):
            idx_issue(i, i)

        def zb(i, _):
            for j in range(D // LANES):
                zbuf[i, pl.ds(j * LANES, LANES)] = jnp.zeros((LANES,), jnp.float32)
            return 0
        lax.fori_loop(0, ZR, zb, 0)

        def zc(t, _):
            pltpu.sync_copy(zbuf, acc.at[pl.ds(s * per_tile_n + t * ZR, ZR)])
            return 0
        lax.fori_loop(0, per_tile_n // ZR, zc, 0)

        plsc.subcore_barrier()

        idx_wait(0, 0)
        gissue(0, 0)
        idx_wait(1, 1)
        gissue(1, 1)

        def quad(j, _):
            i0 = 4 * j
            for u in range(4):
                i = i0 + u
                idx_issue(i + 3, (u + 3) % NB)
                idx_wait(i + 2, (u + 2) % NB)
                gissue(i + 2, (u + 2) % NB)
                gwait(u)
                scat(u)
            return 0
        lax.fori_loop(0, n_chunks // 4 - 1, quad, 0)

        t0 = n_chunks - 4
        for u in range(4):
            i = t0 + u
            if i + 3 < n_chunks:
                idx_issue(i + 3, (u + 3) % NB)
            if i + 2 < n_chunks:
                idx_wait(i + 2, (u + 2) % NB)
                gissue(i + 2, (u + 2) % NB)
            gwait(u)
            scat(u)

        plsc.subcore_barrier()
        pltpu.sync_copy(acc.at[pl.ds(s * per_tile_n, per_tile_n)],
                        out_hbm.at[c, pl.ds(s * per_tile_n, per_tile_n)])

    return pl.kernel(
        body,
        out_type=jax.ShapeDtypeStruct((NC, N_ACC, D), jnp.float32),
        mesh=_mesh(),
        scratch_types=[
            pltpu.VMEM((NB * K,), jnp.int32),
            pltpu.VMEM((NB, K), jnp.int32),
            pltpu.VMEM((NB, K, D), jnp.float32),
            pltpu.VMEM((ZR, D), jnp.float32),
            pltpu.VMEM_SHARED((N_ACC, D), jnp.float32),
            [pltpu.SemaphoreType.DMA] * NB,
            [pltpu.SemaphoreType.DMA] * NB,
        ],
    )


def _dinv(d0_ref, d1_ref):
    deg = d0_ref[...] + d1_ref[...] + 1.0
    return 1.0 / jnp.sqrt(deg)


def _tc_first(x_ref, w_ref, d0_ref, d1_ref, y_ref):
    dinv = _dinv(d0_ref, d1_ref)
    y_ref[...] = jnp.dot(x_ref[...], w_ref[...],
                         preferred_element_type=jnp.float32) * dinv


def _lrelu(x):
    return jnp.where(x >= 0, x, 0.01 * x)


def _tc_mid(a0_ref, a1_ref, y_ref, d0_ref, d1_ref, b_ref, w_ref, o_ref):
    dinv = _dinv(d0_ref, d1_ref)
    pre = (a0_ref[...] + a1_ref[...] + y_ref[...]) * dinv + b_ref[...]
    h = _lrelu(pre)
    o_ref[...] = jnp.dot(h, w_ref[...], preferred_element_type=jnp.float32) * dinv


def _tc_last(a0_ref, a1_ref, y_ref, d0_ref, d1_ref, b_ref, w_ref, bf_ref, o_ref):
    dinv = _dinv(d0_ref, d1_ref)
    pre = (a0_ref[...] + a1_ref[...] + y_ref[...]) * dinv + b_ref[...]
    h = _lrelu(pre)
    o_ref[...] = jnp.dot(h, w_ref[...],
                         preferred_element_type=jnp.float32) + bf_ref[...]


def _full_spec(shape):
    return pl.BlockSpec(shape, lambda i: tuple(0 for _ in shape))


def kernel(inputs, edge_index, W1, b1, W2, b2, Wfc, bfc):
    N, D = inputs.shape
    E = edge_index.shape[1]

    K = 80
    quant = NC * NS * K * 4
    E_pad = ((E + quant - 1) // quant) * quant
    e_per_tile = E_pad // (NC * NS)
    n_chunks = e_per_tile // K
    src_flat = jnp.concatenate(
        [edge_index[0], jnp.zeros((E_pad - E,), jnp.int32)])
    dst_flat = jnp.concatenate(
        [edge_index[1], jnp.full((E_pad - E,), N, jnp.int32)])
    dst = dst_flat.reshape(NC * NS, n_chunks, K)

    N_ACC = ((N + 15) // 16) * 16
    while (N_ACC // NS) % 8 != 0:
        N_ACC += 16

    grid = N // ROW_BLK

    NDEG = ((N + NS * 128 - 1) // (NS * 128)) * (NS * 128)
    deg_fn = _make_deg_kernel(E_pad, NDEG, K)
    degp = deg_fn(dst)
    d0 = degp[0].reshape(NDEG, 1)
    d1 = degp[1].reshape(NDEG, 1)

    dspec = pl.BlockSpec((ROW_BLK, 1), lambda i: (i, 0))
    rspec = pl.BlockSpec((ROW_BLK, D), lambda i: (i, 0))
    wspec = _full_spec((D, D))
    bspec = _full_spec((1, D))

    y1 = pl.pallas_call(
        _tc_first,
        grid=(grid,),
        in_specs=[rspec, wspec, dspec, dspec],
        out_specs=rspec,
        out_shape=jax.ShapeDtypeStruct((N, D), jnp.float32),
    )(inputs, W1, d0, d1)

    agg_fn = _make_agg_kernel(E_pad, D, K, N_ACC)
    aggp1 = agg_fn(y1, src_flat, dst_flat)

    y2 = pl.pallas_call(
        _tc_mid,
        grid=(grid,),
        in_specs=[rspec, rspec, rspec, dspec, dspec, bspec, wspec],
        out_specs=rspec,
        out_shape=jax.ShapeDtypeStruct((N, D), jnp.float32),
    )(aggp1[0], aggp1[1], y1, d0, d1, b1.reshape(1, D), W2)

    aggp2 = agg_fn(y2, src_flat, dst_flat)

    out = pl.pallas_call(
        _tc_last,
        grid=(grid,),
        in_specs=[rspec, rspec, rspec, dspec, dspec, bspec, wspec, bspec],
        out_specs=rspec,
        out_shape=jax.ShapeDtypeStruct((N, D), jnp.float32),
    )(aggp2[0], aggp2[1], y2, d0, d1, b2.reshape(1, D), Wfc, bfc.reshape(1, D))

    return out


# speedup vs baseline: 1.0524x; 1.0524x over previous
import functools

import jax
import jax.numpy as jnp
from jax import lax
from jax.experimental import pallas as pl
from jax.experimental.pallas import tpu as pltpu
from jax.experimental.pallas import tpu_sc as plsc

NC = 2
NS = 16
LANES = 16

ROW_BLK = 2000


def _mesh():
    return plsc.VectorSubcoreMesh(core_axis_name="c", subcore_axis_name="s")


def _make_deg_kernel(E, NDEG, K):
    e_per_tile = E // (NC * NS)
    n_chunks = e_per_tile // K
    per_tile_n = NDEG // NS
    zpad = ((per_tile_n + LANES - 1) // LANES) * LANES

    def body(dst_hbm, out_hbm, dst_buf, ones_v, zvec, acc, sem):
        c = lax.axis_index("c")
        s = lax.axis_index("s")
        wid = c * NS + s

        pltpu.async_copy(dst_hbm.at[wid], dst_buf, sem)

        def zb(i, _):
            zvec[pl.ds(i * LANES, LANES)] = jnp.zeros((LANES,), jnp.float32)
            return 0
        lax.fori_loop(0, zpad // LANES, zb, 0)
        pltpu.sync_copy(zvec.at[pl.ds(0, per_tile_n)],
                        acc.at[pl.ds(s * per_tile_n, per_tile_n)])

        def ob(i, _):
            ones_v[pl.ds(i * LANES, LANES)] = jnp.ones((LANES,), jnp.float32)
            return 0
        lax.fori_loop(0, K // LANES, ob, 0)

        pltpu.make_async_copy(dst_hbm.at[wid], dst_buf, sem).wait()
        plsc.subcore_barrier()

        def chunk(i, _):
            pltpu.sync_copy(ones_v, acc.at[dst_buf.at[i]], add=True)
            return 0
        lax.fori_loop(0, n_chunks, chunk, 0)

        plsc.subcore_barrier()
        pltpu.sync_copy(acc.at[pl.ds(s * per_tile_n, per_tile_n)],
                        out_hbm.at[c, pl.ds(s * per_tile_n, per_tile_n)])

    return pl.kernel(
        body,
        out_type=jax.ShapeDtypeStruct((NC, NDEG), jnp.float32),
        mesh=_mesh(),
        scratch_types=[
            pltpu.VMEM((n_chunks, K), jnp.int32),
            pltpu.VMEM((K,), jnp.float32),
            pltpu.VMEM((zpad,), jnp.float32),
            pltpu.VMEM_SHARED((NDEG,), jnp.float32),
            pltpu.SemaphoreType.DMA,
        ],
    )


def _make_agg_kernel(E, D, K, N_ACC):
    e_per_tile = E // (NC * NS)
    n_chunks = e_per_tile // K
    per_tile_n = N_ACC // NS
    ZR = 8
    NB = 4
    SUP = 8
    n_super = n_chunks // SUP

    def body(y_hbm, src_hbm, dst_hbm, out_hbm,
             src_is, dst_is, rows, zbuf, acc, isems, gsems, ssems):
        c = lax.axis_index("c")
        s = lax.axis_index("s")
        wid = c * NS + s
        ebase = wid * (n_chunks * K)
        wbase = wid * n_super

        def sup_issue(w, p):
            pltpu.async_copy(src_hbm.at[pl.ds(ebase + w * SUP * K, SUP * K)],
                             src_is.at[pl.ds(p * SUP * K, SUP * K)], isems[p])
            pltpu.async_copy(dst_hbm.at[wbase + w],
                             dst_is.at[pl.ds(p * SUP, SUP)], isems[p])

        def sup_wait(w, p):
            pltpu.make_async_copy(
                src_hbm.at[pl.ds(ebase + w * SUP * K, SUP * K)],
                src_is.at[pl.ds(p * SUP * K, SUP * K)], isems[p]).wait()
            pltpu.make_async_copy(
                dst_hbm.at[wbase + w],
                dst_is.at[pl.ds(p * SUP, SUP)], isems[p]).wait()

        def gissue(t, r):
            pltpu.async_copy(y_hbm.at[src_is.at[pl.ds(t * K, K)]],
                             rows.at[r], gsems[r])

        def gwait(t, r):
            pltpu.make_async_copy(y_hbm.at[src_is.at[pl.ds(t * K, K)]],
                                  rows.at[r], gsems[r]).wait()

        def siss(t, r):
            pltpu.async_copy(rows.at[r], acc.at[dst_is.at[t]], ssems[r],
                             add=True)

        def swait(t, r):
            pltpu.make_async_copy(rows.at[r], acc.at[dst_is.at[t]],
                                  ssems[r]).wait()

        sup_issue(0, 0)

        def zb(i, _):
            for j in range(D // LANES):
                zbuf[i, pl.ds(j * LANES, LANES)] = jnp.zeros((LANES,), jnp.float32)
            return 0
        lax.fori_loop(0, ZR, zb, 0)

        def zc(t, _):
            pltpu.sync_copy(zbuf, acc.at[pl.ds(s * per_tile_n + t * ZR, ZR)])
            return 0
        lax.fori_loop(0, per_tile_n // ZR, zc, 0)

        plsc.subcore_barrier()

        sup_wait(0, 0)
        gissue(0, 0)
        gissue(1, 1)

        def block(m, first, last):
            for v in range(16):
                if v == 2:
                    sup_issue(2 * m + 1, 1)
                if v == 10 and not last:
                    sup_issue(2 * m + 2, 0)
                if v == 6:
                    sup_wait(2 * m + 1, 1)
                if v == 14 and not last:
                    sup_wait(2 * m + 2, 0)
                if not (first and v < 2):
                    swait((v - 2) % 16, (v - 2) % NB)
                if not (last and v >= 14):
                    gissue((v + 2) % 16, (v + 2) % NB)
                gwait(v, v % NB)
                siss(v, v % NB)

        block(0, True, False)

        def mid(m, _):
            block(m, False, False)
            return 0
        lax.fori_loop(1, n_super // 2 - 1, mid, 0)

        block(n_super // 2 - 1, False, True)
        swait(14, 2)
        swait(15, 3)

        plsc.subcore_barrier()
        pltpu.sync_copy(acc.at[pl.ds(s * per_tile_n, per_tile_n)],
                        out_hbm.at[c, pl.ds(s * per_tile_n, per_tile_n)])

    return pl.kernel(
        body,
        out_type=jax.ShapeDtypeStruct((NC, N_ACC, D), jnp.float32),
        mesh=_mesh(),
        scratch_types=[
            pltpu.VMEM((2 * SUP * K,), jnp.int32),
            pltpu.VMEM((2 * SUP, K), jnp.int32),
            pltpu.VMEM((NB, K, D), jnp.float32),
            pltpu.VMEM((ZR, D), jnp.float32),
            pltpu.VMEM_SHARED((N_ACC, D), jnp.float32),
            [pltpu.SemaphoreType.DMA] * 2,
            [pltpu.SemaphoreType.DMA] * NB,
            [pltpu.SemaphoreType.DMA] * NB,
        ],
    )


def _dinv(d0_ref, d1_ref):
    deg = d0_ref[...] + d1_ref[...] + 1.0
    return 1.0 / jnp.sqrt(deg)


def _tc_first(x_ref, w_ref, d0_ref, d1_ref, y_ref):
    dinv = _dinv(d0_ref, d1_ref)
    y_ref[...] = jnp.dot(x_ref[...], w_ref[...],
                         preferred_element_type=jnp.float32) * dinv


def _lrelu(x):
    return jnp.where(x >= 0, x, 0.01 * x)


def _tc_mid(a0_ref, a1_ref, y_ref, d0_ref, d1_ref, b_ref, w_ref, o_ref):
    dinv = _dinv(d0_ref, d1_ref)
    pre = (a0_ref[...] + a1_ref[...] + y_ref[...]) * dinv + b_ref[...]
    h = _lrelu(pre)
    o_ref[...] = jnp.dot(h, w_ref[...], preferred_element_type=jnp.float32) * dinv


def _tc_last(a0_ref, a1_ref, y_ref, d0_ref, d1_ref, b_ref, w_ref, bf_ref, o_ref):
    dinv = _dinv(d0_ref, d1_ref)
    pre = (a0_ref[...] + a1_ref[...] + y_ref[...]) * dinv + b_ref[...]
    h = _lrelu(pre)
    o_ref[...] = jnp.dot(h, w_ref[...],
                         preferred_element_type=jnp.float32) + bf_ref[...]


def _full_spec(shape):
    return pl.BlockSpec(shape, lambda i: tuple(0 for _ in shape))


def kernel(inputs, edge_index, W1, b1, W2, b2, Wfc, bfc):
    N, D = inputs.shape
    E = edge_index.shape[1]

    K = 80
    quant = NC * NS * K * 4
    E_pad = ((E + quant - 1) // quant) * quant
    e_per_tile = E_pad // (NC * NS)
    n_chunks = e_per_tile // K
    src_flat = jnp.concatenate(
        [edge_index[0], jnp.zeros((E_pad - E,), jnp.int32)])
    dst_flat = jnp.concatenate(
        [edge_index[1], jnp.full((E_pad - E,), N, jnp.int32)])
    dst = dst_flat.reshape(NC * NS, n_chunks, K)
    dst_sup = dst_flat.reshape(NC * NS * (n_chunks // 8), 8, K)

    N_ACC = ((N + 15) // 16) * 16
    while (N_ACC // NS) % 8 != 0:
        N_ACC += 16

    grid = N // ROW_BLK

    NDEG = ((N + NS * 128 - 1) // (NS * 128)) * (NS * 128)
    deg_fn = _make_deg_kernel(E_pad, NDEG, K)
    degp = deg_fn(dst)
    d0 = degp[0].reshape(NDEG, 1)
    d1 = degp[1].reshape(NDEG, 1)

    dspec = pl.BlockSpec((ROW_BLK, 1), lambda i: (i, 0))
    rspec = pl.BlockSpec((ROW_BLK, D), lambda i: (i, 0))
    wspec = _full_spec((D, D))
    bspec = _full_spec((1, D))

    y1 = pl.pallas_call(
        _tc_first,
        grid=(grid,),
        in_specs=[rspec, wspec, dspec, dspec],
        out_specs=rspec,
        out_shape=jax.ShapeDtypeStruct((N, D), jnp.float32),
    )(inputs, W1, d0, d1)

    agg_fn = _make_agg_kernel(E_pad, D, K, N_ACC)
    aggp1 = agg_fn(y1, src_flat, dst_sup)

    y2 = pl.pallas_call(
        _tc_mid,
        grid=(grid,),
        in_specs=[rspec, rspec, rspec, dspec, dspec, bspec, wspec],
        out_specs=rspec,
        out_shape=jax.ShapeDtypeStruct((N, D), jnp.float32),
    )(aggp1[0], aggp1[1], y1, d0, d1, b1.reshape(1, D), W2)

    aggp2 = agg_fn(y2, src_flat, dst_sup)

    out = pl.pallas_call(
        _tc_last,
        grid=(grid,),
        in_specs=[rspec, rspec, rspec, dspec, dspec, bspec, wspec, bspec],
        out_specs=rspec,
        out_shape=jax.ShapeDtypeStruct((N, D), jnp.float32),
    )(aggp2[0], aggp2[1], y2, d0, d1, b2.reshape(1, D), Wfc, bfc.reshape(1, D))

    return out
